# baseline (speedup 1.0000x reference)
import jax
import jax.numpy as jnp
from jax import lax
from jax.experimental import pallas as pl
from jax.experimental.pallas import tpu as pltpu
from jax.experimental.pallas import tpu_sc as plsc

_E = 320000
_N = 10000
_NP = 10240
_D = 128
_BLK = 2000
_CH = 512
_NCHUNK = _E // _CH
_CHP = 128
_NCHP = _E // _CHP
_NT = 16
_PT128 = _NCHP // _NT
_PAIRS128 = _PT128 // 2
_PT512 = _NCHUNK // _NT
_PAIRS512 = (_PT512 - 1) // 2


def _leaky(x):
    return jnp.where(x >= 0.0, x, 0.01 * x)


def _p1_body(ea, eb, Wa, ba, Wb, bb, ma, mb):
    xa = jnp.dot(ea[...], Wa[...], preferred_element_type=jnp.float32) + ba[...]
    ma[...] = _leaky(xa)
    xb = jnp.dot(eb[...], Wb[...], preferred_element_type=jnp.float32) + bb[...]
    mb[...] = _leaky(xb)


def _p1(e_a, e_b, Wa, ba, Wb, bb):
    blk = pl.BlockSpec((_BLK, _D), lambda i: (i, 0))
    wspec = pl.BlockSpec((_D, _D), lambda i: (0, 0))
    bspec = pl.BlockSpec((1, _D), lambda i: (0, 0))
    return pl.pallas_call(
        _p1_body,
        grid=(_E // _BLK,),
        in_specs=[blk, blk, wspec, bspec, wspec, bspec],
        out_specs=[blk, blk],
        out_shape=[jax.ShapeDtypeStruct((_E, _D), jnp.float32)] * 2,
    )(e_a, e_b, Wa, ba, Wb, bb)


def _sc_mesh():
    return plsc.VectorSubcoreMesh(core_axis_name="c", subcore_axis_name="s")


def _p2a_body(mA, mB, dA, dB, S_a, S_b, mb0, mb1, idx2, Sacc,
              sm0, sm1, ss0, ss1):
    cid = lax.axis_index("c")
    tid = lax.axis_index("s")
    mbufs = (mb0, mb1)
    msems = (sm0, sm1)
    ssems = (ss0, ss1)

    def fill_z(r, _):
        for j in range(_D // 16):
            mb0[r, pl.ds(16 * j, 16)] = jnp.zeros((16,), jnp.float32)
        return 0
    lax.fori_loop(0, 128, fill_z, 0)

    def zero_body(i, _):
        c = tid + _NT * i
        pltpu.sync_copy(mb0, Sacc.at[pl.ds(c * 128, 128)])
        return 0
    lax.fori_loop(0, _NP // 128 // _NT, zero_body, 0)
    plsc.subcore_barrier()

    base = tid * _PT128

    def run(m_hbm, d_hbm):
        def chunk_sync(g):
            pltpu.sync_copy(d_hbm.at[pl.ds(g * _CHP, _CHP)], idx2.at[0])
            pltpu.sync_copy(m_hbm.at[pl.ds(g * _CHP, _CHP)], mb0)
            pltpu.sync_copy(mb0, Sacc.at[idx2.at[0]], add=True)

        def body(i, _):
            hm = [None, None]
            hs = [None, None]
            for b in range(2):
                g = base + 2 * i + b
                pltpu.sync_copy(d_hbm.at[pl.ds(g * _CHP, _CHP)], idx2.at[b])
                hm[b] = pltpu.async_copy(
                    m_hbm.at[pl.ds(g * _CHP, _CHP)], mbufs[b], msems[b])
            for b in range(2):
                hm[b].wait()
                hs[b] = pltpu.async_copy(
                    mbufs[b], Sacc.at[idx2.at[b]], ssems[b], add=True)
            for b in range(2):
                hs[b].wait()
            return 0
        lax.fori_loop(0, _PAIRS128, body, 0)

        @pl.when(tid < _NCHP - _PT128 * _NT)
        def _():
            chunk_sync(_PT128 * _NT + tid)

    @pl.when(cid == 0)
    def _():
        run(mA, dA)

    @pl.when(cid == 1)
    def _():
        run(mB, dB)

    plsc.subcore_barrier()

    def pub_body(i, _):
        c = tid + _NT * i
        sl = pl.ds(c * 128, 128)

        @pl.when(cid == 0)
        def _():
            pltpu.sync_copy(Sacc.at[sl], S_a.at[sl])

        @pl.when(cid == 1)
        def _():
            pltpu.sync_copy(Sacc.at[sl], S_b.at[sl])
        return 0
    lax.fori_loop(0, _NP // 128 // _NT, pub_body, 0)


def _p2a(m_a, m_b, dst_a, dst_b):
    f = pl.kernel(
        _p2a_body,
        out_type=[jax.ShapeDtypeStruct((_NP, _D), jnp.float32)] * 2,
        mesh=_sc_mesh(),
        scratch_types=[
            pltpu.VMEM((_CHP, _D), jnp.float32),
            pltpu.VMEM((_CHP, _D), jnp.float32),
            pltpu.VMEM((2, 128), jnp.int32),
            pltpu.VMEM_SHARED((_NP, _D), jnp.float32),
            pltpu.SemaphoreType.DMA,
            pltpu.SemaphoreType.DMA,
            pltpu.SemaphoreType.DMA,
            pltpu.SemaphoreType.DMA,
        ],
    )
    return f(m_a, m_b, dst_a, dst_b)


def _p2b_body(dA, dB, C_a, C_b, ones, idx2, Cacc, cs0, cs1):
    cid = lax.axis_index("c")
    tid = lax.axis_index("s")
    csems = (cs0, cs1)

    def fill_z(r, _):
        for j in range(_D // 16):
            ones[r, pl.ds(16 * j, 16)] = jnp.zeros((16,), jnp.float32)
        return 0
    lax.fori_loop(0, 128, fill_z, 0)

    def zero_body(i, _):
        c = tid + _NT * i
        pltpu.sync_copy(ones, Cacc.at[pl.ds(c * 128, 128)])
        return 0
    lax.fori_loop(0, _NP // 128 // _NT, zero_body, 0)

    def fill_one(r, _):
        for j in range(_D // 16):
            ones[r, pl.ds(16 * j, 16)] = jnp.ones((16,), jnp.float32)
        return 0
    lax.fori_loop(0, 128, fill_one, 0)
    plsc.subcore_barrier()

    base = tid * _PT512

    def run(d_hbm):
        def chunk_sync(g):
            for j in range(_CH // 128):
                pltpu.sync_copy(
                    d_hbm.at[pl.ds(g * _CH + j * 128, 128)], idx2.at[j])
            for j in range(_CH // 128):
                pltpu.sync_copy(ones, Cacc.at[idx2.at[j]], add=True)

        def body(i, _):
            hs = []
            for b in range(2):
                g = base + 2 * i + b
                for j in range(_CH // 128):
                    k = b * (_CH // 128) + j
                    pltpu.sync_copy(
                        d_hbm.at[pl.ds(g * _CH + j * 128, 128)], idx2.at[k])
                    hs.append(pltpu.async_copy(
                        ones, Cacc.at[idx2.at[k]], csems[b], add=True))
            for h in hs:
                h.wait()
            return 0
        lax.fori_loop(0, _PAIRS512, body, 0)
        chunk_sync(base + 2 * _PAIRS512)

        @pl.when(tid == _NT - 1)
        def _():
            chunk_sync(_NCHUNK - 1)

    @pl.when(cid == 0)
    def _():
        run(dA)

    @pl.when(cid == 1)
    def _():
        run(dB)

    plsc.subcore_barrier()

    def pub_body(i, _):
        c = tid + _NT * i
        sl = pl.ds(c * 128, 128)

        @pl.when(cid == 0)
        def _():
            pltpu.sync_copy(Cacc.at[sl], C_a.at[sl])

        @pl.when(cid == 1)
        def _():
            pltpu.sync_copy(Cacc.at[sl], C_b.at[sl])
        return 0
    lax.fori_loop(0, _NP // 128 // _NT, pub_body, 0)


def _p2b(dst_a, dst_b):
    f = pl.kernel(
        _p2b_body,
        out_type=[jax.ShapeDtypeStruct((_NP, _D), jnp.float32)] * 2,
        mesh=_sc_mesh(),
        scratch_types=[
            pltpu.VMEM((128, _D), jnp.float32),
            pltpu.VMEM((2 * (_CH // 128), 128), jnp.int32),
            pltpu.VMEM_SHARED((_NP, _D), jnp.float32),
            pltpu.SemaphoreType.DMA,
            pltpu.SemaphoreType.DMA,
        ],
    )
    return f(dst_a, dst_b)


def _p3_body(Sa, Ca, Sb, Cb, Wam, Wal, Wbm, Wbl, Qas, Qad, Qbs, Qbd):
    rel_a = Sa[...] / jnp.maximum(Ca[...][:, 0:1], 1.0)
    rel_b = Sb[...] / jnp.maximum(Cb[...][:, 0:1], 1.0)
    Qas[...] = jnp.dot(rel_a, Wam[...], preferred_element_type=jnp.float32)
    Qad[...] = jnp.dot(rel_b, Wal[...], preferred_element_type=jnp.float32)
    Qbs[...] = jnp.dot(rel_a, Wbm[...], preferred_element_type=jnp.float32)
    Qbd[...] = jnp.dot(rel_b, Wbl[...], preferred_element_type=jnp.float32)


def _p3(Sa, Ca, Sb, Cb, Wam, Wal, Wbm, Wbl):
    nb = 2048
    blk = pl.BlockSpec((nb, _D), lambda i: (i, 0))
    wspec = pl.BlockSpec((_D, _D), lambda i: (0, 0))
    return pl.pallas_call(
        _p3_body,
        grid=(_NP // nb,),
        in_specs=[blk, blk, blk, blk, wspec, wspec, wspec, wspec],
        out_specs=[blk] * 4,
        out_shape=[jax.ShapeDtypeStruct((_NP, _D), jnp.float32)] * 4,
    )(Sa, Ca, Sb, Cb, Wam, Wal, Wbm, Wbl)


def _p4_body(Qas, Qad, Qbs, Qbd, sA, dA, sB, dB,
             Gsa, Gsb, bS0, bS1, bS2, bD0, bD1, bD2, idx4, idx1d,
             gS0, gS1, gS2, gD0, gD1, gD2, w0, w1, w2):
    cid = lax.axis_index("c")
    tid = lax.axis_index("s")
    bufS = (bS0, bS1, bS2)
    bufD = (bD0, bD1, bD2)
    gsS = (gS0, gS1, gS2)
    gsD = (gD0, gD1, gD2)
    wsem = (w0, w1, w2)
    base = tid * _PT128

    def add_rows(dst, src):
        def row(r, _):
            for j in range(_D // 16):
                sl = pl.ds(16 * j, 16)
                dst[r, sl] = dst[r, sl] + src[r, sl]
            return 0
        lax.fori_loop(0, _CHP, row, 0)

    def run(Qs, Qd, s_hbm, d_hbm, G):
        def chunk_sync(g):
            pltpu.sync_copy(s_hbm.at[pl.ds(g * _CHP, _CHP)], idx4.at[0])
            pltpu.sync_copy(d_hbm.at[pl.ds(g * _CHP, _CHP)], idx4.at[3])
            pltpu.async_copy(Qs.at[idx4.at[0]], bS0, gS0).wait()
            pltpu.async_copy(Qd.at[idx4.at[3]], bD0, gD0).wait()
            add_rows(bS0, bD0)
            pltpu.sync_copy(bS0, G.at[pl.ds(g * _CHP, _CHP)])

        def body(i, _):
            g0 = base + 3 * i
            his = pltpu.async_copy(
                s_hbm.at[pl.ds(g0 * _CHP, 3 * _CHP)], idx1d.at[0], gS0)
            hid = pltpu.async_copy(
                d_hbm.at[pl.ds(g0 * _CHP, 3 * _CHP)], idx1d.at[1], gD0)
            his.wait()
            hid.wait()
            for k in range(3 * _CHP // 16):
                idx4[k // 8, pl.ds(16 * (k % 8), 16)] = \
                    idx1d[0, pl.ds(16 * k, 16)]
                idx4[3 + k // 8, pl.ds(16 * (k % 8), 16)] = \
                    idx1d[1, pl.ds(16 * k, 16)]
            hgS = [None] * 3
            hgD = [None] * 3
            for b in range(3):
                hgS[b] = pltpu.async_copy(Qs.at[idx4.at[b]], bufS[b], gsS[b])
                hgD[b] = pltpu.async_copy(Qd.at[idx4.at[3 + b]], bufD[b],
                                          gsD[b])
            hw = []
            for b in range(3):
                g = base + 3 * i + b
                hgS[b].wait()
                hgD[b].wait()
                add_rows(bufS[b], bufD[b])
                hw.append(pltpu.async_copy(
                    bufS[b], G.at[pl.ds(g * _CHP, _CHP)], wsem[b]))
            for h in hw:
                h.wait()
            return 0
        lax.fori_loop(0, _PT128 // 3, body, 0)

        @pl.when(tid < _NCHP - _PT128 * _NT)
        def _():
            chunk_sync(_PT128 * _NT + tid)

    @pl.when(cid == 0)
    def _():
        run(Qas, Qad, sA, dA, Gsa)

    @pl.when(cid == 1)
    def _():
        run(Qbs, Qbd, sB, dB, Gsb)


def _p4(Qas, Qad, Qbs, Qbd, sA, dA, sB, dB):
    f = pl.kernel(
        _p4_body,
        out_type=[jax.ShapeDtypeStruct((_E, _D), jnp.float32)] * 2,
        mesh=_sc_mesh(),
        scratch_types=[
            pltpu.VMEM((_CHP, _D), jnp.float32),
            pltpu.VMEM((_CHP, _D), jnp.float32),
            pltpu.VMEM((_CHP, _D), jnp.float32),
            pltpu.VMEM((_CHP, _D), jnp.float32),
            pltpu.VMEM((_CHP, _D), jnp.float32),
            pltpu.VMEM((_CHP, _D), jnp.float32),
            pltpu.VMEM((6, 128), jnp.int32),
            pltpu.VMEM((2, 3 * _CHP), jnp.int32),
            pltpu.SemaphoreType.DMA,
            pltpu.SemaphoreType.DMA,
            pltpu.SemaphoreType.DMA,
            pltpu.SemaphoreType.DMA,
            pltpu.SemaphoreType.DMA,
            pltpu.SemaphoreType.DMA,
            pltpu.SemaphoreType.DMA,
            pltpu.SemaphoreType.DMA,
            pltpu.SemaphoreType.DMA,
        ],
    )
    return f(Qas, Qad, Qbs, Qbd, sA, dA, sB, dB)


def _p5_body(ea, eb, Ga, Gb, Wha, bha, Whb, bhb, out):
    oa = jnp.dot(ea[...], Wha[...], preferred_element_type=jnp.float32)
    out[0] = oa + bha[...] + Ga[...]
    ob = jnp.dot(eb[...], Whb[...], preferred_element_type=jnp.float32)
    out[1] = ob + bhb[...] + Gb[...]


def _p5(e_a, e_b, Ga, Gb, Wha, bha, Whb, bhb):
    blk = pl.BlockSpec((_BLK, _D), lambda i: (i, 0))
    wspec = pl.BlockSpec((_D, _D), lambda i: (0, 0))
    bspec = pl.BlockSpec((1, _D), lambda i: (0, 0))
    oblk = pl.BlockSpec((2, _BLK, _D), lambda i: (0, i, 0))
    return pl.pallas_call(
        _p5_body,
        grid=(_E // _BLK,),
        in_specs=[blk, blk, blk, blk, wspec, bspec, wspec, bspec],
        out_specs=oblk,
        out_shape=jax.ShapeDtypeStruct((2, _E, _D), jnp.float32),
    )(e_a, e_b, Ga, Gb, Wha, bha, Whb, bhb)


def kernel(e_a, e_b, edge_index_a, edge_index_b,
           W_dense_a, b_dense_a, W_dense_b, b_dense_b,
           W_e_a, b_e_a, W_e_b, b_e_b):
    srcA = edge_index_a[0]
    dstA = edge_index_a[1]
    srcB = edge_index_b[0]
    dstB = edge_index_b[1]

    ba = b_dense_a.reshape(1, _D)
    bb = b_dense_b.reshape(1, _D)
    bha = b_e_a.reshape(1, _D)
    bhb = b_e_b.reshape(1, _D)

    C_a, C_b = _p2b(dstA, dstB)
    m_a, m_b = _p1(e_a, e_b, W_dense_a, ba, W_dense_b, bb)
    S_a, S_b = _p2a(m_a, m_b, dstA, dstB)
    Qas, Qad, Qbs, Qbd = _p3(
        S_a, C_a, S_b, C_b,
        W_e_a[_D:2 * _D], W_e_a[2 * _D:3 * _D],
        W_e_b[_D:2 * _D], W_e_b[2 * _D:3 * _D],
    )
    Ga, Gb = _p4(Qas, Qad, Qbs, Qbd, srcA, dstA, srcB, dstB)
    return _p5(e_a, e_b, Ga, Gb, W_e_a[:_D], bha, W_e_b[:_D], bhb)


# speedup vs baseline: 1.0167x; 1.0167x over previous
import jax
import jax.numpy as jnp
from jax import lax
from jax.experimental import pallas as pl
from jax.experimental.pallas import tpu as pltpu
from jax.experimental.pallas import tpu_sc as plsc

_E = 320000
_N = 10000
_NP = 10240
_D = 128
_BLK = 2000
_CH = 512
_NCHUNK = _E // _CH
_CHP = 128
_NCHP = _E // _CHP
_NT = 16
_PT128 = _NCHP // _NT
_PAIRS128 = _PT128 // 2
_PT512 = _NCHUNK // _NT
_PAIRS512 = (_PT512 - 1) // 2


def _leaky(x):
    return jnp.where(x >= 0.0, x, 0.01 * x)


def _p1_body(ea, eb, Wa, ba, Wb, bb, ma, mb):
    xa = jnp.dot(ea[...], Wa[...], preferred_element_type=jnp.float32) + ba[...]
    ma[...] = _leaky(xa)
    xb = jnp.dot(eb[...], Wb[...], preferred_element_type=jnp.float32) + bb[...]
    mb[...] = _leaky(xb)


def _p1(e_a, e_b, Wa, ba, Wb, bb):
    blk = pl.BlockSpec((_BLK, _D), lambda i: (i, 0))
    wspec = pl.BlockSpec((_D, _D), lambda i: (0, 0))
    bspec = pl.BlockSpec((1, _D), lambda i: (0, 0))
    return pl.pallas_call(
        _p1_body,
        grid=(_E // _BLK,),
        in_specs=[blk, blk, wspec, bspec, wspec, bspec],
        out_specs=[blk, blk],
        out_shape=[jax.ShapeDtypeStruct((_E, _D), jnp.float32)] * 2,
    )(e_a, e_b, Wa, ba, Wb, bb)


def _sc_mesh():
    return plsc.VectorSubcoreMesh(core_axis_name="c", subcore_axis_name="s")


def _p2a_body(mA, mB, dA, dB, S_a, S_b, mb0, mb1, idx2, Sacc,
              sm0, sm1, ss0, ss1):
    cid = lax.axis_index("c")
    tid = lax.axis_index("s")
    mbufs = (mb0, mb1)
    msems = (sm0, sm1)
    ssems = (ss0, ss1)

    def fill_z(r, _):
        for j in range(_D // 16):
            mb0[r, pl.ds(16 * j, 16)] = jnp.zeros((16,), jnp.float32)
        return 0
    lax.fori_loop(0, 128, fill_z, 0)

    def zero_body(i, _):
        c = tid + _NT * i
        pltpu.sync_copy(mb0, Sacc.at[pl.ds(c * 128, 128)])
        return 0
    lax.fori_loop(0, _NP // 128 // _NT, zero_body, 0)
    plsc.subcore_barrier()

    base = tid * _PT128

    def run(m_hbm, d_hbm):
        def chunk_sync(g):
            pltpu.sync_copy(d_hbm.at[pl.ds(g * _CHP, _CHP)], idx2.at[0])
            pltpu.sync_copy(m_hbm.at[pl.ds(g * _CHP, _CHP)], mb0)
            pltpu.sync_copy(mb0, Sacc.at[idx2.at[0]], add=True)

        def body(i, _):
            hm = [None, None]
            hs = [None, None]
            for b in range(2):
                g = base + 2 * i + b
                pltpu.sync_copy(d_hbm.at[pl.ds(g * _CHP, _CHP)], idx2.at[b])
                hm[b] = pltpu.async_copy(
                    m_hbm.at[pl.ds(g * _CHP, _CHP)], mbufs[b], msems[b])
            for b in range(2):
                hm[b].wait()
                hs[b] = pltpu.async_copy(
                    mbufs[b], Sacc.at[idx2.at[b]], ssems[b], add=True)
            for b in range(2):
                hs[b].wait()
            return 0
        lax.fori_loop(0, _PAIRS128, body, 0)

        @pl.when(tid < _NCHP - _PT128 * _NT)
        def _():
            chunk_sync(_PT128 * _NT + tid)

    @pl.when(cid == 0)
    def _():
        run(mA, dA)

    @pl.when(cid == 1)
    def _():
        run(mB, dB)

    plsc.subcore_barrier()

    def pub_body(i, _):
        c = tid + _NT * i
        sl = pl.ds(c * 128, 128)

        @pl.when(cid == 0)
        def _():
            pltpu.sync_copy(Sacc.at[sl], S_a.at[sl])

        @pl.when(cid == 1)
        def _():
            pltpu.sync_copy(Sacc.at[sl], S_b.at[sl])
        return 0
    lax.fori_loop(0, _NP // 128 // _NT, pub_body, 0)


def _p2a(m_a, m_b, dst_a, dst_b):
    f = pl.kernel(
        _p2a_body,
        out_type=[jax.ShapeDtypeStruct((_NP, _D), jnp.float32)] * 2,
        mesh=_sc_mesh(),
        scratch_types=[
            pltpu.VMEM((_CHP, _D), jnp.float32),
            pltpu.VMEM((_CHP, _D), jnp.float32),
            pltpu.VMEM((2, 128), jnp.int32),
            pltpu.VMEM_SHARED((_NP, _D), jnp.float32),
            pltpu.SemaphoreType.DMA,
            pltpu.SemaphoreType.DMA,
            pltpu.SemaphoreType.DMA,
            pltpu.SemaphoreType.DMA,
        ],
    )
    return f(m_a, m_b, dst_a, dst_b)


def _p2b_body(dA, dB, C_a, C_b, ones, idx2, Cacc, cs0, cs1):
    cid = lax.axis_index("c")
    tid = lax.axis_index("s")
    csems = (cs0, cs1)

    def fill_z(r, _):
        for j in range(_D // 16):
            ones[r, pl.ds(16 * j, 16)] = jnp.zeros((16,), jnp.float32)
        return 0
    lax.fori_loop(0, 128, fill_z, 0)

    def zero_body(i, _):
        c = tid + _NT * i
        pltpu.sync_copy(ones, Cacc.at[pl.ds(c * 128, 128)])
        return 0
    lax.fori_loop(0, _NP // 128 // _NT, zero_body, 0)

    def fill_one(r, _):
        for j in range(_D // 16):
            ones[r, pl.ds(16 * j, 16)] = jnp.ones((16,), jnp.float32)
        return 0
    lax.fori_loop(0, 128, fill_one, 0)
    plsc.subcore_barrier()

    base = tid * _PT512

    def run(d_hbm):
        def chunk_sync(g):
            for j in range(_CH // 128):
                pltpu.sync_copy(
                    d_hbm.at[pl.ds(g * _CH + j * 128, 128)], idx2.at[j])
            for j in range(_CH // 128):
                pltpu.sync_copy(ones, Cacc.at[idx2.at[j]], add=True)

        def body(i, _):
            hs = []
            for b in range(2):
                g = base + 2 * i + b
                for j in range(_CH // 128):
                    k = b * (_CH // 128) + j
                    pltpu.sync_copy(
                        d_hbm.at[pl.ds(g * _CH + j * 128, 128)], idx2.at[k])
                    hs.append(pltpu.async_copy(
                        ones, Cacc.at[idx2.at[k]], csems[b], add=True))
            for h in hs:
                h.wait()
            return 0
        lax.fori_loop(0, _PAIRS512, body, 0)
        chunk_sync(base + 2 * _PAIRS512)

        @pl.when(tid == _NT - 1)
        def _():
            chunk_sync(_NCHUNK - 1)

    @pl.when(cid == 0)
    def _():
        run(dA)

    @pl.when(cid == 1)
    def _():
        run(dB)

    plsc.subcore_barrier()

    def pub_body(i, _):
        c = tid + _NT * i
        sl = pl.ds(c * 128, 128)

        @pl.when(cid == 0)
        def _():
            pltpu.sync_copy(Cacc.at[sl], C_a.at[sl])

        @pl.when(cid == 1)
        def _():
            pltpu.sync_copy(Cacc.at[sl], C_b.at[sl])
        return 0
    lax.fori_loop(0, _NP // 128 // _NT, pub_body, 0)


def _p2b(dst_a, dst_b):
    f = pl.kernel(
        _p2b_body,
        out_type=[jax.ShapeDtypeStruct((_NP, _D), jnp.float32)] * 2,
        mesh=_sc_mesh(),
        scratch_types=[
            pltpu.VMEM((128, _D), jnp.float32),
            pltpu.VMEM((2 * (_CH // 128), 128), jnp.int32),
            pltpu.VMEM_SHARED((_NP, _D), jnp.float32),
            pltpu.SemaphoreType.DMA,
            pltpu.SemaphoreType.DMA,
        ],
    )
    return f(dst_a, dst_b)


def _p3_body(Sa, Ca, Sb, Cb, Wam, Wal, Wbm, Wbl, Qas, Qad, Qbs, Qbd):
    rel_a = Sa[...] / jnp.maximum(Ca[...][:, 0:1], 1.0)
    rel_b = Sb[...] / jnp.maximum(Cb[...][:, 0:1], 1.0)
    Qas[...] = jnp.dot(rel_a, Wam[...], preferred_element_type=jnp.float32)
    Qad[...] = jnp.dot(rel_b, Wal[...], preferred_element_type=jnp.float32)
    Qbs[...] = jnp.dot(rel_a, Wbm[...], preferred_element_type=jnp.float32)
    Qbd[...] = jnp.dot(rel_b, Wbl[...], preferred_element_type=jnp.float32)


def _p3(Sa, Ca, Sb, Cb, Wam, Wal, Wbm, Wbl):
    nb = 2048
    blk = pl.BlockSpec((nb, _D), lambda i: (i, 0))
    wspec = pl.BlockSpec((_D, _D), lambda i: (0, 0))
    return pl.pallas_call(
        _p3_body,
        grid=(_NP // nb,),
        in_specs=[blk, blk, blk, blk, wspec, wspec, wspec, wspec],
        out_specs=[blk] * 4,
        out_shape=[jax.ShapeDtypeStruct((_NP, _D), jnp.float32)] * 4,
    )(Sa, Ca, Sb, Cb, Wam, Wal, Wbm, Wbl)


def _p4_body(Qas, Qad, Qbs, Qbd, sA, dA, sB, dB,
             Gsa, Gsb, bS0, bS1, bS2, bD0, bD1, bD2, idx4,
             gS0, gS1, gS2, gD0, gD1, gD2, w0, w1, w2):
    cid = lax.axis_index("c")
    tid = lax.axis_index("s")
    bufS = (bS0, bS1, bS2)
    bufD = (bD0, bD1, bD2)
    gsS = (gS0, gS1, gS2)
    gsD = (gD0, gD1, gD2)
    wsem = (w0, w1, w2)
    base = tid * _PT128

    def add_rows(dst, src):
        def row(r, _):
            for j in range(_D // 16):
                sl = pl.ds(16 * j, 16)
                dst[r, sl] = dst[r, sl] + src[r, sl]
            return 0
        lax.fori_loop(0, _CHP, row, 0)

    def run(Qs, Qd, s_hbm, d_hbm, G):
        def chunk_sync(g):
            pltpu.sync_copy(s_hbm.at[pl.ds(g * _CHP, _CHP)], idx4.at[0])
            pltpu.sync_copy(d_hbm.at[pl.ds(g * _CHP, _CHP)], idx4.at[3])
            pltpu.async_copy(Qs.at[idx4.at[0]], bS0, gS0).wait()
            pltpu.async_copy(Qd.at[idx4.at[3]], bD0, gD0).wait()
            add_rows(bS0, bD0)
            pltpu.sync_copy(bS0, G.at[pl.ds(g * _CHP, _CHP)])

        def body(i, _):
            hgS = [None] * 3
            hgD = [None] * 3
            for b in range(3):
                g = base + 3 * i + b
                pltpu.sync_copy(s_hbm.at[pl.ds(g * _CHP, _CHP)], idx4.at[b])
                pltpu.sync_copy(d_hbm.at[pl.ds(g * _CHP, _CHP)],
                                idx4.at[3 + b])
                hgS[b] = pltpu.async_copy(Qs.at[idx4.at[b]], bufS[b], gsS[b])
                hgD[b] = pltpu.async_copy(Qd.at[idx4.at[3 + b]], bufD[b],
                                          gsD[b])
            hw = []
            for b in range(3):
                g = base + 3 * i + b
                hgS[b].wait()
                hgD[b].wait()
                add_rows(bufS[b], bufD[b])
                hw.append(pltpu.async_copy(
                    bufS[b], G.at[pl.ds(g * _CHP, _CHP)], wsem[b]))
            for h in hw:
                h.wait()
            return 0
        lax.fori_loop(0, _PT128 // 3, body, 0)

        @pl.when(tid < _NCHP - _PT128 * _NT)
        def _():
            chunk_sync(_PT128 * _NT + tid)

    @pl.when(cid == 0)
    def _():
        run(Qas, Qad, sA, dA, Gsa)

    @pl.when(cid == 1)
    def _():
        run(Qbs, Qbd, sB, dB, Gsb)


def _p4(Qas, Qad, Qbs, Qbd, sA, dA, sB, dB):
    f = pl.kernel(
        _p4_body,
        out_type=[jax.ShapeDtypeStruct((_E, _D), jnp.float32)] * 2,
        mesh=_sc_mesh(),
        scratch_types=[
            pltpu.VMEM((_CHP, _D), jnp.float32),
            pltpu.VMEM((_CHP, _D), jnp.float32),
            pltpu.VMEM((_CHP, _D), jnp.float32),
            pltpu.VMEM((_CHP, _D), jnp.float32),
            pltpu.VMEM((_CHP, _D), jnp.float32),
            pltpu.VMEM((_CHP, _D), jnp.float32),
            pltpu.VMEM((6, 128), jnp.int32),
            pltpu.SemaphoreType.DMA,
            pltpu.SemaphoreType.DMA,
            pltpu.SemaphoreType.DMA,
            pltpu.SemaphoreType.DMA,
            pltpu.SemaphoreType.DMA,
            pltpu.SemaphoreType.DMA,
            pltpu.SemaphoreType.DMA,
            pltpu.SemaphoreType.DMA,
            pltpu.SemaphoreType.DMA,
        ],
    )
    return f(Qas, Qad, Qbs, Qbd, sA, dA, sB, dB)


def _p5_body(ea, eb, Ga, Gb, Wha, bha, Whb, bhb, out):
    oa = jnp.dot(ea[...], Wha[...], preferred_element_type=jnp.float32)
    out[0] = oa + bha[...] + Ga[...]
    ob = jnp.dot(eb[...], Whb[...], preferred_element_type=jnp.float32)
    out[1] = ob + bhb[...] + Gb[...]


def _p5(e_a, e_b, Ga, Gb, Wha, bha, Whb, bhb):
    blk = pl.BlockSpec((_BLK, _D), lambda i: (i, 0))
    wspec = pl.BlockSpec((_D, _D), lambda i: (0, 0))
    bspec = pl.BlockSpec((1, _D), lambda i: (0, 0))
    oblk = pl.BlockSpec((2, _BLK, _D), lambda i: (0, i, 0))
    return pl.pallas_call(
        _p5_body,
        grid=(_E // _BLK,),
        in_specs=[blk, blk, blk, blk, wspec, bspec, wspec, bspec],
        out_specs=oblk,
        out_shape=jax.ShapeDtypeStruct((2, _E, _D), jnp.float32),
    )(e_a, e_b, Ga, Gb, Wha, bha, Whb, bhb)


def kernel(e_a, e_b, edge_index_a, edge_index_b,
           W_dense_a, b_dense_a, W_dense_b, b_dense_b,
           W_e_a, b_e_a, W_e_b, b_e_b):
    srcA = edge_index_a[0]
    dstA = edge_index_a[1]
    srcB = edge_index_b[0]
    dstB = edge_index_b[1]

    ba = b_dense_a.reshape(1, _D)
    bb = b_dense_b.reshape(1, _D)
    bha = b_e_a.reshape(1, _D)
    bhb = b_e_b.reshape(1, _D)

    C_a, C_b = _p2b(dstA, dstB)
    m_a, m_b = _p1(e_a, e_b, W_dense_a, ba, W_dense_b, bb)
    S_a, S_b = _p2a(m_a, m_b, dstA, dstB)
    Qas, Qad, Qbs, Qbd = _p3(
        S_a, C_a, S_b, C_b,
        W_e_a[_D:2 * _D], W_e_a[2 * _D:3 * _D],
        W_e_b[_D:2 * _D], W_e_b[2 * _D:3 * _D],
    )
    Ga, Gb = _p4(Qas, Qad, Qbs, Qbd, srcA, dstA, srcB, dstB)
    return _p5(e_a, e_b, Ga, Gb, W_e_a[:_D], bha, W_e_b[:_D], bhb)
